# baseline (speedup 1.0000x reference)
import math

import jax
import jax.numpy as jnp
from jax import lax
from jax.experimental import pallas as pl
from jax.experimental.pallas import tpu as pltpu
from jax.experimental.pallas import tpu_sc as plsc

N_TOKEN = 1000000
D_MODEL = 64
SCALE = math.sqrt(D_MODEL)

NC, NS = 2, 16
NW = NC * NS
N_BATCH = 4096
N_TOK = 200
LANES = 128
DR = D_MODEL // 8
NBUF = 4
LOOKAHEAD = 3


def _emb_kernel(xt_hbm, lut_hbm, out_hbm, idx_v, in_buf, tr_buf, *sems):
    g_sems = sems[:NBUF]
    s_sems = sems[NBUF:]
    w = lax.axis_index("s") * NC + lax.axis_index("c")
    pltpu.sync_copy(xt_hbm.at[:, pl.ds(w * LANES, LANES)], idx_v)

    iota2 = lax.broadcasted_iota(jnp.int32, (16,), 0) * 2
    d_ev = [iota2 + c * 32 for c in range(D_MODEL // 32)]
    d_od = [iota2 + (c * 32 + 1) for c in range(D_MODEL // 32)]

    def gather_start(t, b):
        pltpu.async_copy(lut_hbm.at[idx_v.at[t]], in_buf.at[b], g_sems[b])

    def gather_wait(t, b):
        pltpu.make_async_copy(lut_hbm.at[idx_v.at[t]], in_buf.at[b],
                              g_sems[b]).wait()

    def tr_slice(b, dr):
        return tr_buf.at[b, pl.ds(dr * 8, 8), pl.ds(0, LANES)]

    def store_start(t, b):
        for dr in range(DR):
            pltpu.async_copy(tr_slice(b, dr), out_hbm.at[t, dr, w],
                             s_sems[b])

    def store_wait(b):
        for dr in range(DR):
            pltpu.make_async_copy(tr_slice(b, dr), out_hbm.at[0, dr, w],
                                  s_sems[b]).wait()

    def transpose_scale(b):
        @plsc.parallel_loop(0, LANES, unroll=4)
        def _r(r):
            r_idx = jnp.full((16,), r, jnp.int32)
            for c in range(D_MODEL // 32):
                v = in_buf[b, r, pl.ds(c * 32, 32)]
                ev, od = plsc.unpack(v, format=plsc.PackFormat.INTERLEAVED)
                plsc.store_scatter(tr_buf.at[b], [d_ev[c], r_idx],
                                   ev * SCALE)
                plsc.store_scatter(tr_buf.at[b], [d_od[c], r_idx],
                                   od * SCALE)

    def process(t, b, issue_t=None, wait_store=True):
        if issue_t is not None:
            gather_start(issue_t, (b + LOOKAHEAD) % NBUF)
        gather_wait(t, b)
        if wait_store:
            store_wait(b)
        transpose_scale(b)
        store_start(t, b)

    for t in range(LOOKAHEAD):
        gather_start(t, t % NBUF)

    for b in range(NBUF):
        process(b, b, issue_t=b + LOOKAHEAD, wait_store=False)

    @pl.loop(NBUF, N_TOK - NBUF, step=NBUF)
    def _grp(t0):
        for b in range(NBUF):
            process(t0 + b, b, issue_t=t0 + b + LOOKAHEAD)

    for b in range(NBUF):
        t = N_TOK - NBUF + b
        process(t, b,
                issue_t=(t + LOOKAHEAD) if t + LOOKAHEAD < N_TOK else None)

    for b in range(NBUF):
        store_wait(b)


@jax.jit
def _emb(xt, lut):
    mesh = plsc.VectorSubcoreMesh(core_axis_name="c", subcore_axis_name="s")
    f = pl.kernel(
        _emb_kernel,
        out_type=jax.ShapeDtypeStruct((N_TOK, DR, NW, 8, LANES),
                                      jnp.float32),
        mesh=mesh,
        compiler_params=pltpu.CompilerParams(use_tc_tiling_on_sc=False,
                                             needs_layout_passes=False),
        scratch_types=(
            [pltpu.VMEM((N_TOK, LANES), jnp.int32),
             pltpu.VMEM((NBUF, LANES, D_MODEL), jnp.bfloat16),
             pltpu.VMEM((NBUF, D_MODEL, LANES + 8), jnp.float32)]
            + [pltpu.SemaphoreType.DMA] * (2 * NBUF)
        ),
    )
    return f(xt, lut)


def kernel(x, lut):
    xt = x.T.astype(jnp.int32)
    out5 = _emb(xt, lut.astype(jnp.bfloat16))
    return out5.transpose(2, 4, 0, 1, 3).reshape(N_BATCH, N_TOK, D_MODEL)


# speedup vs baseline: 1.2639x; 1.2639x over previous
import math

import jax
import jax.numpy as jnp
from jax import lax
from jax.experimental import pallas as pl
from jax.experimental.pallas import tpu as pltpu
from jax.experimental.pallas import tpu_sc as plsc

N_TOKEN = 1000000
D_MODEL = 64
SCALE = math.sqrt(D_MODEL)

NC, NS = 2, 16
NW = NC * NS
N_BATCH = 4096
N_TOK = 200
LANES = 128
DR = D_MODEL // 8
NBUF = 4
LOOKAHEAD = 3


def _emb_kernel(xt_hbm, lut_hbm, out_hbm, idx_v, in_buf, tr_buf, *sems):
    g_sems = sems[:NBUF]
    s_sems = sems[NBUF:]
    w = lax.axis_index("s") * NC + lax.axis_index("c")
    pltpu.sync_copy(xt_hbm.at[:, pl.ds(w * LANES, LANES)], idx_v)

    d_ids = [lax.broadcasted_iota(jnp.int32, (16,), 0) + c * 16
             for c in range(D_MODEL // 16)]

    def gather_start(t, b):
        pltpu.async_copy(lut_hbm.at[idx_v.at[t]], in_buf.at[b], g_sems[b])

    def gather_wait(t, b):
        pltpu.make_async_copy(lut_hbm.at[idx_v.at[t]], in_buf.at[b],
                              g_sems[b]).wait()

    def tr_slice(b, dr):
        return tr_buf.at[b, pl.ds(dr * 8, 8), pl.ds(0, LANES)]

    def store_start(t, b):
        for dr in range(DR):
            pltpu.async_copy(tr_slice(b, dr), out_hbm.at[t, dr, w],
                             s_sems[b])

    def store_wait(b):
        for dr in range(DR):
            pltpu.make_async_copy(tr_slice(b, dr), out_hbm.at[0, dr, w],
                                  s_sems[b]).wait()

    def transpose_scale(b):
        @plsc.parallel_loop(0, LANES, unroll=4)
        def _r(r):
            r_idx = jnp.full((16,), r, jnp.int32)
            for c in range(D_MODEL // 16):
                vals = in_buf[b, r, pl.ds(c * 16, 16)] * SCALE
                plsc.store_scatter(tr_buf.at[b], [d_ids[c], r_idx], vals)

    def process(t, b, issue_t=None, wait_store=True):
        if issue_t is not None:
            gather_start(issue_t, (b + LOOKAHEAD) % NBUF)
        gather_wait(t, b)
        if wait_store:
            store_wait(b)
        transpose_scale(b)
        store_start(t, b)

    for t in range(LOOKAHEAD):
        gather_start(t, t % NBUF)

    for b in range(NBUF):
        process(b, b, issue_t=b + LOOKAHEAD, wait_store=False)

    @pl.loop(NBUF, N_TOK - NBUF, step=NBUF)
    def _grp(t0):
        for b in range(NBUF):
            process(t0 + b, b, issue_t=t0 + b + LOOKAHEAD)

    for b in range(NBUF):
        t = N_TOK - NBUF + b
        process(t, b,
                issue_t=(t + LOOKAHEAD) if t + LOOKAHEAD < N_TOK else None)

    for b in range(NBUF):
        store_wait(b)


@jax.jit
def _emb(xt, lut):
    mesh = plsc.VectorSubcoreMesh(core_axis_name="c", subcore_axis_name="s")
    f = pl.kernel(
        _emb_kernel,
        out_type=jax.ShapeDtypeStruct((N_TOK, DR, NW, 8, LANES),
                                      jnp.float32),
        mesh=mesh,
        compiler_params=pltpu.CompilerParams(use_tc_tiling_on_sc=False,
                                             needs_layout_passes=False),
        scratch_types=(
            [pltpu.VMEM((N_TOK, LANES), jnp.int32),
             pltpu.VMEM((NBUF, LANES, D_MODEL), jnp.float32),
             pltpu.VMEM((NBUF, D_MODEL, LANES + 8), jnp.float32)]
            + [pltpu.SemaphoreType.DMA] * (2 * NBUF)
        ),
    )
    return f(xt, lut)


def kernel(x, lut):
    xt = x.T.astype(jnp.int32)
    out5 = _emb(xt, lut)
    return out5.transpose(2, 4, 0, 1, 3).reshape(N_BATCH, N_TOK, D_MODEL)
